# Initial kernel scaffold; baseline (speedup 1.0000x reference)
import jax
import jax.numpy as jnp
from jax.experimental import pallas as pl


def kernel(input_ids, table):
    raise NotImplementedError("write your pallas kernel here")


# speedup vs baseline: 1.4314x; 1.4314x over previous
import functools

import jax
import jax.numpy as jnp
from jax import lax
from jax.experimental import pallas as pl
from jax.experimental.pallas import tpu as pltpu
from jax.experimental.pallas import tpu_sc as plsc


def _emb_kernel(n_rows, d, n_workers, num_cores, chunk):
    n_per_w = n_rows // n_workers
    n_chunks = n_per_w // chunk

    mesh = plsc.VectorSubcoreMesh(core_axis_name="c", subcore_axis_name="s")

    @functools.partial(
        pl.kernel,
        mesh=mesh,
        out_type=jax.ShapeDtypeStruct((n_rows, d), jnp.float32),
        scratch_types=[
            pltpu.VMEM((n_per_w,), jnp.int32),
            pltpu.VMEM((chunk, d), jnp.float32),
            pltpu.SemaphoreType.DMA,
        ],
    )
    def emb(idx_hbm, table_hbm, out_hbm, idx_v, rows_v, sem):
        wid = lax.axis_index("s") * num_cores + lax.axis_index("c")
        base = wid * n_per_w
        pltpu.sync_copy(idx_hbm.at[pl.ds(base, n_per_w)], idx_v)

        def body(i, carry):
            off = i * chunk
            pltpu.async_copy(
                table_hbm.at[idx_v.at[pl.ds(off, chunk)]], rows_v, sem
            ).wait()
            pltpu.sync_copy(rows_v, out_hbm.at[pl.ds(base + off, chunk)])
            return carry

        lax.fori_loop(0, n_chunks, body, 0)

    return emb


def kernel(input_ids, table):
    b, s = input_ids.shape
    v, d = table.shape
    n = b * s
    idx = input_ids.reshape(n).astype(jnp.int32)
    info = plsc.get_sparse_core_info()
    nw = info.num_cores * info.num_subcores
    emb = _emb_kernel(n, d, nw, info.num_cores, chunk=16)
    out = emb(idx, table)
    return out.reshape(b, s, d)


# speedup vs baseline: 1.6631x; 1.1619x over previous
import functools

import jax
import jax.numpy as jnp
from jax import lax
from jax.experimental import pallas as pl
from jax.experimental.pallas import tpu as pltpu
from jax.experimental.pallas import tpu_sc as plsc


def _emb_kernel(n_rows, d, n_workers, num_cores, chunk):
    n_per_w = n_rows // n_workers
    n_chunks = n_per_w // chunk
    assert n_chunks % 2 == 0

    mesh = plsc.VectorSubcoreMesh(core_axis_name="c", subcore_axis_name="s")

    @functools.partial(
        pl.kernel,
        mesh=mesh,
        out_type=jax.ShapeDtypeStruct((n_rows, d), jnp.float32),
        scratch_types=[
            pltpu.VMEM((n_per_w,), jnp.int32),
            pltpu.VMEM((2, chunk, d), jnp.float32),
            pltpu.SemaphoreType.DMA,
            pltpu.SemaphoreType.DMA,
            pltpu.SemaphoreType.DMA,
            pltpu.SemaphoreType.DMA,
        ],
    )
    def emb(idx_hbm, table_hbm, out_hbm, idx_v, rows_v, si0, si1, so0, so1):
        sin = (si0, si1)
        sout = (so0, so1)
        wid = lax.axis_index("s") * num_cores + lax.axis_index("c")
        base = wid * n_per_w
        pltpu.sync_copy(idx_hbm.at[pl.ds(base, n_per_w)], idx_v)

        def gather(i, b):
            return pltpu.make_async_copy(
                table_hbm.at[idx_v.at[pl.ds(i * chunk, chunk)]],
                rows_v.at[b],
                sin[b],
            )

        def put(i, b):
            return pltpu.make_async_copy(
                rows_v.at[b],
                out_hbm.at[pl.ds(base + i * chunk, chunk)],
                sout[b],
            )

        gather(0, 0).start()

        def body(k, carry):
            for b in range(2):
                i = 2 * k + b
                nb = 1 - b

                @pl.when(i >= 1)
                def _():
                    put(i - 1, nb).wait()

                @pl.when(i + 1 < n_chunks)
                def _():
                    gather(i + 1, nb).start()

                gather(i, b).wait()
                put(i, b).start()
            return carry

        lax.fori_loop(0, n_chunks // 2, body, 0)
        put(n_chunks - 1, 1).wait()

    return emb


def kernel(input_ids, table):
    b, s = input_ids.shape
    v, d = table.shape
    n = b * s
    idx = input_ids.reshape(n).astype(jnp.int32)
    info = plsc.get_sparse_core_info()
    nw = info.num_cores * info.num_subcores
    emb = _emb_kernel(n, d, nw, info.num_cores, chunk=16)
    out = emb(idx, table)
    return out.reshape(b, s, d)


# speedup vs baseline: 1.6878x; 1.0148x over previous
import functools

import jax
import jax.numpy as jnp
from jax import lax
from jax.experimental import pallas as pl
from jax.experimental.pallas import tpu as pltpu
from jax.experimental.pallas import tpu_sc as plsc

_NBUF = 4


def _emb_kernel(n_rows, d, n_workers, num_cores, chunk):
    n_per_w = n_rows // n_workers
    n_chunks = n_per_w // chunk
    assert n_chunks % _NBUF == 0 and n_chunks >= 2 * _NBUF

    mesh = plsc.VectorSubcoreMesh(core_axis_name="c", subcore_axis_name="s")

    @functools.partial(
        pl.kernel,
        mesh=mesh,
        out_type=jax.ShapeDtypeStruct((n_rows, d), jnp.float32),
        scratch_types=[
            pltpu.VMEM((n_per_w,), jnp.int32),
            pltpu.VMEM((_NBUF, chunk, d), jnp.float32),
        ]
        + [pltpu.SemaphoreType.DMA] * (2 * _NBUF),
    )
    def emb(idx_hbm, table_hbm, out_hbm, idx_v, rows_v, *sems):
        sin = sems[:_NBUF]
        sout = sems[_NBUF:]
        wid = lax.axis_index("s") * num_cores + lax.axis_index("c")
        base = wid * n_per_w
        pltpu.sync_copy(idx_hbm.at[pl.ds(base, n_per_w)], idx_v)

        def gather(i, b):
            return pltpu.make_async_copy(
                table_hbm.at[idx_v.at[pl.ds(i * chunk, chunk)]],
                rows_v.at[b],
                sin[b],
            )

        def put(i, b):
            return pltpu.make_async_copy(
                rows_v.at[b],
                out_hbm.at[pl.ds(base + i * chunk, chunk)],
                sout[b],
            )

        gather(0, 0).start()
        gather(1, 1).start()

        def body(k, carry):
            for b in range(_NBUF):
                i = _NBUF * k + b
                nb = (b + 2) % _NBUF

                @pl.when(i >= 2)
                def _():
                    put(i - 2, nb).wait()

                @pl.when(i + 2 < n_chunks)
                def _():
                    gather(i + 2, nb).start()

                gather(i, b).wait()
                put(i, b).start()
            return carry

        lax.fori_loop(0, n_chunks // _NBUF, body, 0)
        put(n_chunks - 2, (n_chunks - 2) % _NBUF).wait()
        put(n_chunks - 1, (n_chunks - 1) % _NBUF).wait()

    return emb


def kernel(input_ids, table):
    b, s = input_ids.shape
    v, d = table.shape
    n = b * s
    idx = input_ids.reshape(n).astype(jnp.int32)
    info = plsc.get_sparse_core_info()
    nw = info.num_cores * info.num_subcores
    emb = _emb_kernel(n, d, nw, info.num_cores, chunk=8)
    out = emb(idx, table)
    return out.reshape(b, s, d)
